# baseline (speedup 1.0000x reference)
import functools

import jax
import jax.numpy as jnp
from jax import lax
from jax.experimental import pallas as pl
from jax.experimental.pallas import tpu as pltpu
from jax.experimental.pallas import tpu_sc as plsc

B, S, V, D = 4, 8192, 100000, 768
CHUNK = 32
NBUF = 4


def _make_lookup():
    info = plsc.get_sparse_core_info()
    nc, ns = info.num_cores, info.num_subcores
    nw = nc * ns
    n_slabs = nw // B
    slab = S // n_slabs
    assert slab % CHUNK == 0
    n_chunks = slab // CHUNK

    mesh = plsc.VectorSubcoreMesh(core_axis_name="c", subcore_axis_name="s")

    @functools.partial(
        pl.kernel,
        mesh=mesh,
        out_type=jax.ShapeDtypeStruct((S, B, D), jnp.float32),
        scratch_types=[
            pltpu.VMEM((n_chunks, CHUNK), jnp.int32),
        ] + [pltpu.VMEM((CHUNK, D), jnp.float32)] * NBUF
          + [pltpu.SemaphoreType.DMA] * (2 * NBUF),
    )
    def lookup(table_hbm, idx_hbm, out_hbm, idx_v, *bufs_sems):
        bufs = bufs_sems[:NBUF]
        gsems = bufs_sems[NBUF:2 * NBUF]
        wsems = bufs_sems[2 * NBUF:]
        wid = lax.axis_index("s") * nc + lax.axis_index("c")
        b = wid // n_slabs
        sb = wid % n_slabs
        s_base = sb * slab
        pltpu.sync_copy(idx_hbm.at[b, sb], idx_v)

        def gather(c, slot):
            return pltpu.async_copy(table_hbm.at[idx_v.at[c]], bufs[slot],
                                    gsems[slot])

        def write(c, slot):
            return pltpu.async_copy(
                bufs[slot],
                out_hbm.at[pl.ds(s_base + c * CHUNK, CHUNK), b, :],
                wsems[slot])

        gathers = [None] * NBUF
        writes = [None] * NBUF
        for j in range(NBUF - 1):
            gathers[j] = gather(j, j)
        for c in range(n_chunks):
            slot = c % NBUF
            g = c + NBUF - 1
            if g < n_chunks:
                gs = g % NBUF
                if writes[gs] is not None:
                    writes[gs].wait()
                gathers[gs] = gather(g, gs)
            gathers[slot].wait()
            writes[slot] = write(c, slot)
        for w in writes:
            if w is not None:
                w.wait()

    return lookup, n_slabs, n_chunks


def kernel(enc_tokens, dec_tokens, enc_attn_mask, dec_attn_mask,
           enc_dec_attn_mask, dec_labels, emb_table):
    lookup, n_slabs, n_chunks = _make_lookup()
    idx = enc_tokens.astype(jnp.int32).reshape(B, n_slabs, n_chunks, CHUNK)
    return lookup(emb_table, idx)


# speedup vs baseline: 1.0142x; 1.0142x over previous
import functools

import jax
import jax.numpy as jnp
from jax import lax
from jax.experimental import pallas as pl
from jax.experimental.pallas import tpu as pltpu
from jax.experimental.pallas import tpu_sc as plsc

B, S, V, D = 4, 8192, 100000, 768
CHUNK = 64
NBUF = 2


def _make_lookup():
    info = plsc.get_sparse_core_info()
    nc, ns = info.num_cores, info.num_subcores
    nw = nc * ns
    n_slabs = nw // B
    slab = S // n_slabs
    assert slab % CHUNK == 0
    n_chunks = slab // CHUNK

    mesh = plsc.VectorSubcoreMesh(core_axis_name="c", subcore_axis_name="s")

    @functools.partial(
        pl.kernel,
        mesh=mesh,
        out_type=jax.ShapeDtypeStruct((S, B, D), jnp.float32),
        scratch_types=[
            pltpu.VMEM((slab,), jnp.int32),
        ] + [pltpu.VMEM((CHUNK, D), jnp.float32)] * NBUF
          + [pltpu.SemaphoreType.DMA] * (2 * NBUF),
    )
    def lookup(table_hbm, tok_hbm, out_hbm, idx_v, *bufs_sems):
        bufs = bufs_sems[:NBUF]
        gsems = bufs_sems[NBUF:2 * NBUF]
        wsems = bufs_sems[2 * NBUF:]
        wid = lax.axis_index("s") * nc + lax.axis_index("c")
        b = wid // n_slabs
        sb = wid % n_slabs
        s_base = sb * slab
        pltpu.sync_copy(tok_hbm.at[b, pl.ds(s_base, slab)], idx_v)

        def gather(c, slot):
            return pltpu.async_copy(
                table_hbm.at[idx_v.at[pl.ds(c * CHUNK, CHUNK)]], bufs[slot],
                gsems[slot])

        def write(c, slot):
            return pltpu.async_copy(
                bufs[slot],
                out_hbm.at[pl.ds(s_base + c * CHUNK, CHUNK), b, :],
                wsems[slot])

        gathers = [None] * NBUF
        writes = [None] * NBUF
        for j in range(NBUF - 1):
            gathers[j] = gather(j, j)
        for c in range(n_chunks):
            slot = c % NBUF
            g = c + NBUF - 1
            if g < n_chunks:
                gs = g % NBUF
                if writes[gs] is not None:
                    writes[gs].wait()
                gathers[gs] = gather(g, gs)
            gathers[slot].wait()
            writes[slot] = write(c, slot)
        for w in writes:
            if w is not None:
                w.wait()

    return lookup, n_slabs, n_chunks


def kernel(enc_tokens, dec_tokens, enc_attn_mask, dec_attn_mask,
           enc_dec_attn_mask, dec_labels, emb_table):
    lookup, n_slabs, n_chunks = _make_lookup()
    return lookup(emb_table, enc_tokens.astype(jnp.int32))
